# baseline (speedup 1.0000x reference)
import functools

import jax
import jax.numpy as jnp
from jax import lax
from jax.experimental import pallas as pl
from jax.experimental.pallas import tpu as pltpu
from jax.experimental.pallas import tpu_sc as plsc

_IDXW = 128
_K = 4
_DP = 128


_PAD_BLK = 10000


def _pad_body(src_ref, dst_ref):
    blk = src_ref[...]
    dst_ref[:, : blk.shape[-1]] = blk


@functools.lru_cache(maxsize=None)
def _build_pad(V, D):
    return pl.pallas_call(
        _pad_body,
        grid=(V // _PAD_BLK,),
        in_specs=[pl.BlockSpec((_PAD_BLK, D), lambda i: (i, 0))],
        out_specs=pl.BlockSpec((_PAD_BLK, _DP), lambda i: (i, 0)),
        out_shape=jax.ShapeDtypeStruct((V, _DP), jnp.float32),
    )


@functools.lru_cache(maxsize=None)
def _build_gather(N, V, D):
    info = plsc.get_sparse_core_info()
    NC, NS = info.num_cores, info.num_subcores
    NW = NC * NS
    n_rows = N // _IDXW
    rows_per_w = n_rows // NW
    n_groups = rows_per_w // _K
    assert N % _IDXW == 0 and n_rows % NW == 0 and rows_per_w % _K == 0

    mesh = plsc.VectorSubcoreMesh(core_axis_name="c", subcore_axis_name="s")

    @functools.partial(
        pl.kernel,
        out_type=jax.ShapeDtypeStruct((n_rows, _IDXW, _DP), jnp.float32),
        mesh=mesh,
        scratch_types=[
            pltpu.VMEM((_K, _IDXW), jnp.int32),
            pltpu.VMEM((_K, _IDXW, _DP), jnp.float32),
            pltpu.SemaphoreType.DMA,
        ],
    )
    def k(idx_hbm, table_hbm, out_hbm, idx_v, rows_v, gsem):
        wid = lax.axis_index("s") * NC + lax.axis_index("c")
        row_base = wid * rows_per_w

        def body(g, _):
            base = row_base + g * _K
            pltpu.sync_copy(idx_hbm.at[pl.ds(base, _K)], idx_v)
            cps = [
                pltpu.async_copy(
                    table_hbm.at[idx_v.at[j]], rows_v.at[j], gsem
                )
                for j in range(_K)
            ]
            for cp in cps:
                cp.wait()
            pltpu.sync_copy(rows_v, out_hbm.at[pl.ds(base, _K)])
            return 0

        lax.fori_loop(0, n_groups, body, 0)

    return k


def kernel(X, wv):
    B, H = X.shape
    V, D = wv.shape
    N = B * H
    idx2d = X.reshape(N // _IDXW, _IDXW)
    wv_p = _build_pad(V, D)(wv)
    out = _build_gather(N, V, D)(idx2d, wv_p)
    return out[:, :, :D].reshape(B, H, D)


# speedup vs baseline: 1.0298x; 1.0298x over previous
import functools

import jax
import jax.numpy as jnp
from jax import lax
from jax.experimental import pallas as pl
from jax.experimental.pallas import tpu as pltpu
from jax.experimental.pallas import tpu_sc as plsc

_IDXW = 128
_K = 2
_DP = 128


_PAD_BLK = 10000


def _pad_body(src_ref, dst_ref):
    blk = src_ref[...]
    dst_ref[:, : blk.shape[-1]] = blk


@functools.lru_cache(maxsize=None)
def _build_pad(V, D):
    return pl.pallas_call(
        _pad_body,
        grid=(V // _PAD_BLK,),
        in_specs=[pl.BlockSpec((_PAD_BLK, D), lambda i: (i, 0))],
        out_specs=pl.BlockSpec((_PAD_BLK, _DP), lambda i: (i, 0)),
        out_shape=jax.ShapeDtypeStruct((V, _DP), jnp.float32),
    )


@functools.lru_cache(maxsize=None)
def _build_gather(N, V, D):
    info = plsc.get_sparse_core_info()
    NC, NS = info.num_cores, info.num_subcores
    NW = NC * NS
    n_rows = N // _IDXW
    rows_per_w = n_rows // NW
    n_groups = rows_per_w // _K
    assert N % _IDXW == 0 and n_rows % NW == 0 and rows_per_w % _K == 0
    assert n_groups % 2 == 0

    mesh = plsc.VectorSubcoreMesh(core_axis_name="c", subcore_axis_name="s")

    @functools.partial(
        pl.kernel,
        out_type=jax.ShapeDtypeStruct((n_rows, _IDXW, _DP), jnp.float32),
        mesh=mesh,
        scratch_types=[
            pltpu.VMEM((2, _K, _IDXW), jnp.int32),
            pltpu.VMEM((2, _K, _IDXW, _DP), jnp.float32),
            pltpu.SemaphoreType.DMA,
        ],
    )
    def k(idx_hbm, table_hbm, out_hbm, idx_v, rows_v, gsem):
        wid = lax.axis_index("s") * NC + lax.axis_index("c")
        row_base = wid * rows_per_w

        def stage_and_fire(g, b):
            base = row_base + g * _K
            pltpu.sync_copy(idx_hbm.at[pl.ds(base, _K)], idx_v.at[b])
            for j in range(_K):
                pltpu.async_copy(
                    table_hbm.at[idx_v.at[b].at[j]],
                    rows_v.at[b].at[j],
                    gsem,
                )

        stage_and_fire(0, 0)

        def outer(g2, _):
            for b in range(2):
                g = g2 * 2 + b
                nb = 1 - b

                @pl.when(g + 1 < n_groups)
                def _():
                    stage_and_fire(g + 1, nb)

                base = row_base + g * _K
                pltpu.make_async_copy(
                    out_hbm.at[pl.ds(base, _K)], rows_v.at[b], gsem
                ).wait()
                pltpu.sync_copy(rows_v.at[b], out_hbm.at[pl.ds(base, _K)])
            return 0

        lax.fori_loop(0, n_groups // 2, outer, 0)

    return k


def kernel(X, wv):
    B, H = X.shape
    V, D = wv.shape
    N = B * H
    idx2d = X.reshape(N // _IDXW, _IDXW)
    wv_p = _build_pad(V, D)(wv)
    out = _build_gather(N, V, D)(idx2d, wv_p)
    return out[:, :, :D].reshape(B, H, D)


# speedup vs baseline: 1.0302x; 1.0004x over previous
import functools

import jax
import jax.numpy as jnp
from jax import lax
from jax.experimental import pallas as pl
from jax.experimental.pallas import tpu as pltpu
from jax.experimental.pallas import tpu_sc as plsc

_IDXW = 128
_K = 2
_DP = 128


_PAD_BLK = 10000


def _pad_body(src_ref, dst_ref):
    blk = src_ref[...]
    dst_ref[:, : blk.shape[-1]] = blk


@functools.lru_cache(maxsize=None)
def _build_pad(V, D):
    return pl.pallas_call(
        _pad_body,
        grid=(V // _PAD_BLK,),
        in_specs=[pl.BlockSpec((_PAD_BLK, D), lambda i: (i, 0))],
        out_specs=pl.BlockSpec((_PAD_BLK, _DP), lambda i: (i, 0)),
        out_shape=jax.ShapeDtypeStruct((V, _DP), jnp.float32),
    )


@functools.lru_cache(maxsize=None)
def _build_gather(N, V, D):
    info = plsc.get_sparse_core_info()
    NC, NS = info.num_cores, info.num_subcores
    NW = NC * NS
    n_rows = N // _IDXW
    rows_per_w = n_rows // NW
    n_groups = rows_per_w // _K
    assert N % _IDXW == 0 and n_rows % NW == 0 and rows_per_w % _K == 0
    assert n_groups % 2 == 0

    mesh = plsc.VectorSubcoreMesh(core_axis_name="c", subcore_axis_name="s")

    @functools.partial(
        pl.kernel,
        out_type=jax.ShapeDtypeStruct((n_rows, _IDXW, _DP), jnp.float32),
        mesh=mesh,
        scratch_types=[
            pltpu.VMEM((2, _K, _IDXW), jnp.int32),
            pltpu.VMEM((2, _K, _IDXW, _DP), jnp.float32),
            pltpu.SemaphoreType.DMA,
            pltpu.SemaphoreType.DMA,
        ],
    )
    def k(idx_hbm, table_hbm, out_hbm, idx_v, rows_v, gsem, osem):
        wid = lax.axis_index("s") * NC + lax.axis_index("c")
        row_base = wid * rows_per_w

        def stage_and_fire(g, b):
            base = row_base + g * _K
            pltpu.sync_copy(idx_hbm.at[pl.ds(base, _K)], idx_v.at[b])
            for j in range(_K):
                pltpu.async_copy(
                    table_hbm.at[idx_v.at[b].at[j]],
                    rows_v.at[b].at[j],
                    gsem,
                )

        def drain_out(b):
            pltpu.make_async_copy(
                rows_v.at[b], out_hbm.at[pl.ds(row_base, _K)], osem
            ).wait()

        stage_and_fire(0, 0)

        def outer(g2, _):
            for b in range(2):
                g = g2 * 2 + b
                nb = 1 - b

                @pl.when(g + 1 < n_groups)
                def _():
                    @pl.when(g >= 1)
                    def _():
                        drain_out(nb)

                    stage_and_fire(g + 1, nb)

                base = row_base + g * _K
                pltpu.make_async_copy(
                    out_hbm.at[pl.ds(base, _K)], rows_v.at[b], gsem
                ).wait()
                pltpu.async_copy(rows_v.at[b], out_hbm.at[pl.ds(base, _K)], osem)
            return 0

        lax.fori_loop(0, n_groups // 2, outer, 0)
        drain_out(0)
        drain_out(1)

    return k


def kernel(X, wv):
    B, H = X.shape
    V, D = wv.shape
    N = B * H
    idx2d = X.reshape(N // _IDXW, _IDXW)
    wv_p = _build_pad(V, D)(wv)
    out = _build_gather(N, V, D)(idx2d, wv_p)
    return out[:, :, :D].reshape(B, H, D)
